# baseline (speedup 1.0000x reference)
import jax
import jax.numpy as jnp
from jax.experimental import pallas as pl


def _pos_kernel(col_ref, row_ref, out_ref):
    d, w = col_ref.shape
    h = row_ref.shape[1]
    c_blk = out_ref.shape[1]
    ci = pl.program_id(1)
    n_col = d // c_blk

    @pl.when(ci < n_col)
    def _col():
        sl = col_ref[pl.ds(ci * c_blk, c_blk), :]
        out_ref[0] = jnp.broadcast_to(sl[:, None, :], (c_blk, h, w))

    @pl.when(ci >= n_col)
    def _row():
        sl = row_ref[pl.ds((ci - n_col) * c_blk, c_blk), :]
        out_ref[0] = jnp.broadcast_to(sl[:, :, None], (c_blk, h, w))


def kernel(x, row_w, col_w):
    b = x.shape[0]
    h, w = x.shape[-2], x.shape[-1]
    d = row_w.shape[-1]
    col_t = col_w[:w].T
    row_t = row_w[:h].T

    c_blk = 32
    n_c = 2 * d // c_blk

    out = pl.pallas_call(
        _pos_kernel,
        grid=(b, n_c),
        in_specs=[
            pl.BlockSpec((d, w), lambda bi, ci: (0, 0)),
            pl.BlockSpec((d, h), lambda bi, ci: (0, 0)),
        ],
        out_specs=pl.BlockSpec((1, c_blk, h, w), lambda bi, ci: (bi, ci, 0, 0)),
        out_shape=jax.ShapeDtypeStruct((b, 2 * d, h, w), jnp.float32),
    )(col_t, row_t)
    return out.reshape(b, 2 * d, h * w)


# speedup vs baseline: 2.1734x; 2.1734x over previous
import functools

import jax
import jax.numpy as jnp
from jax import lax
from jax.experimental import pallas as pl
from jax.experimental.pallas import tpu as pltpu
from jax.experimental.pallas import tpu_sc as plsc

_B = 4
_H = 224
_W = 384
_D = 128
_L = 16
_NW = 32
_CPW = 2 * _D // _NW


def _pos_body(col_hbm, row_hbm, out_hbm, pat_v, rw_v, plane_v):
    wid = lax.axis_index("s") * 2 + lax.axis_index("c")
    kpr = _W // _L

    for t in range(_CPW):
        c = wid * _CPW + t

        @pl.when(c < _D)
        def _col(c=c):
            pltpu.sync_copy(col_hbm.at[c], pat_v)
            pat = [pat_v[pl.ds(_L * k, _L)] for k in range(kpr)]

            def body(r, carry):
                base = r * _W
                for k in range(kpr):
                    plane_v[pl.ds(base + _L * k, _L)] = pat[k]
                return carry

            lax.fori_loop(0, _H, body, 0)
            for b in range(_B):
                pltpu.sync_copy(plane_v, out_hbm.at[b, c])

        @pl.when(c >= _D)
        def _row(c=c):
            pltpu.sync_copy(row_hbm.at[c - _D], rw_v)

            def body(r, carry):
                v = rw_v[pl.ds(r * _L, _L)]
                base = r * _W
                for k in range(kpr):
                    plane_v[pl.ds(base + _L * k, _L)] = v
                return carry

            lax.fori_loop(0, _H, body, 0)
            for b in range(_B):
                pltpu.sync_copy(plane_v, out_hbm.at[b, c])


def kernel(x, row_w, col_w):
    b = x.shape[0]
    h, w = x.shape[-2], x.shape[-1]
    d = row_w.shape[-1]
    col_t = col_w[:w].T
    row_t = jnp.repeat(row_w[:h].T, _L, axis=1)

    mesh = plsc.VectorSubcoreMesh(core_axis_name="c", subcore_axis_name="s")
    run = functools.partial(
        pl.kernel,
        mesh=mesh,
        out_type=jax.ShapeDtypeStruct((b, 2 * d, h * w), jnp.float32),
        scratch_types=[
            pltpu.VMEM((w,), jnp.float32),
            pltpu.VMEM((h * _L,), jnp.float32),
            pltpu.VMEM((h * w,), jnp.float32),
        ],
    )(_pos_body)
    return run(col_t, row_t)


# speedup vs baseline: 2.4846x; 1.1432x over previous
import functools

import jax
import jax.numpy as jnp
from jax import lax
from jax.experimental import pallas as pl
from jax.experimental.pallas import tpu as pltpu
from jax.experimental.pallas import tpu_sc as plsc

_B = 4
_H = 224
_W = 384
_D = 128
_L = 16
_NW = 32
_CPW = 2 * _D // _NW
_HROWS = _H // 2
_HALF = _HROWS * _W


def _pos_body(col_hbm, row_hbm, out_hbm, pat_v, rw_v, buf0_v, buf1_v, sem0, sem1):
    wid = lax.axis_index("s") * 2 + lax.axis_index("c")
    kpr = _W // _L
    bufs = (buf0_v, buf1_v)
    sems = (sem0, sem1)
    inflight = [[], []]

    for slot in range(2 * _CPW):
        t, hh = divmod(slot, 2)
        c = wid * _CPW + t
        nb = slot % 2
        buf, sem = bufs[nb], sems[nb]

        for cp in inflight[nb]:
            cp.wait()
        inflight[nb] = []

        @pl.when(c < _D)
        def _col(c=c, buf=buf):
            if hh == 0:
                pltpu.sync_copy(col_hbm.at[c], pat_v)
            pat = [pat_v[pl.ds(_L * k, _L)] for k in range(kpr)]

            def body(r, carry):
                base = r * _W
                for k in range(kpr):
                    buf[pl.ds(base + _L * k, _L)] = pat[k]
                return carry

            lax.fori_loop(0, _HROWS, body, 0)

        @pl.when(c >= _D)
        def _row(c=c, buf=buf, hh=hh):
            if hh == 0:
                pltpu.sync_copy(row_hbm.at[c - _D], rw_v)

            def body(r, carry):
                v = rw_v[pl.ds((hh * _HROWS + r) * _L, _L)]
                base = r * _W
                for k in range(kpr):
                    buf[pl.ds(base + _L * k, _L)] = v
                return carry

            lax.fori_loop(0, _HROWS, body, 0)

        inflight[nb] = [
            pltpu.async_copy(buf, out_hbm.at[b, c, pl.ds(hh * _HALF, _HALF)], sem)
            for b in range(_B)
        ]

    for pend in inflight:
        for cp in pend:
            cp.wait()


def kernel(x, row_w, col_w):
    b = x.shape[0]
    h, w = x.shape[-2], x.shape[-1]
    d = row_w.shape[-1]
    col_t = col_w[:w].T
    row_t = jnp.repeat(row_w[:h].T, _L, axis=1)

    mesh = plsc.VectorSubcoreMesh(core_axis_name="c", subcore_axis_name="s")
    run = functools.partial(
        pl.kernel,
        mesh=mesh,
        out_type=jax.ShapeDtypeStruct((b, 2 * d, h * w), jnp.float32),
        scratch_types=[
            pltpu.VMEM((w,), jnp.float32),
            pltpu.VMEM((h * _L,), jnp.float32),
            pltpu.VMEM((_HALF,), jnp.float32),
            pltpu.VMEM((_HALF,), jnp.float32),
            pltpu.SemaphoreType.DMA,
            pltpu.SemaphoreType.DMA,
        ],
    )(_pos_body)
    return run(col_t, row_t)
